# Initial kernel scaffold; baseline (speedup 1.0000x reference)
import functools

import jax
import jax.numpy as jnp
from jax import lax
from jax.experimental import pallas as pl
from jax.experimental.pallas import tpu as pltpu
from jax.experimental.pallas import tpu_sc as plsc

_HIDDEN = 128
_LANES = 16
_CHUNK = 128


def _scalar_rsqrt(v):
    i = lax.bitcast_convert_type(v, jnp.int32)
    i = jnp.int32(0x5F3759DF) - (i >> 1)
    y = lax.bitcast_convert_type(i, jnp.float32)
    for _ in range(3):
        y = y * (1.5 - 0.5 * v * y * y)
    return y


def _make_sc_kernel(tok, hidden, num_workers):
    per_w = tok // num_workers
    n_chunks = per_w // _CHUNK
    n_vregs = hidden // _LANES
    mesh = plsc.VectorSubcoreMesh(core_axis_name="c", subcore_axis_name="s")

    @functools.partial(
        pl.kernel,
        mesh=mesh,
        out_type=jax.ShapeDtypeStruct((tok, hidden), jnp.float32),
        scratch_types=[
            pltpu.VMEM((_CHUNK,), jnp.int32),
            pltpu.VMEM((_CHUNK,), jnp.int32),
            pltpu.VMEM((_CHUNK, hidden), jnp.float32),
            pltpu.VMEM((_CHUNK, hidden), jnp.float32),
            pltpu.VMEM((hidden,), jnp.float32),
            pltpu.VMEM((hidden,), jnp.float32),
            pltpu.SemaphoreType.DMA,
            pltpu.SemaphoreType.DMA,
        ],
    )
    def body(ids_hbm, cidx_hbm, wemb_hbm, ptab_hbm, g_hbm, b_hbm, out_hbm,
             idx_v, cidx_v, rows_v, pt_v, g_v, b_v, sem_w, sem_p):
        wid = lax.axis_index("s") * 2 + lax.axis_index("c")
        base0 = wid * per_w
        pltpu.sync_copy(g_hbm, g_v)
        pltpu.sync_copy(b_hbm, b_v)

        def chunk_body(ci, carry):
            base = base0 + ci * _CHUNK
            pltpu.sync_copy(ids_hbm.at[pl.ds(base, _CHUNK)], idx_v)
            pltpu.sync_copy(cidx_hbm.at[pl.ds(base, _CHUNK)], cidx_v)
            cp_w = pltpu.async_copy(wemb_hbm.at[idx_v], rows_v, sem_w)
            cp_p = pltpu.async_copy(ptab_hbm.at[cidx_v], pt_v, sem_p)
            cp_w.wait()
            cp_p.wait()

            def tok_body(t, c2):
                x = [rows_v[t, pl.ds(h * _LANES, _LANES)]
                     + pt_v[t, pl.ds(h * _LANES, _LANES)]
                     for h in range(n_vregs)]
                s = x[0]
                q = x[0] * x[0]
                for h in range(1, n_vregs):
                    s = s + x[h]
                    q = q + x[h] * x[h]
                inv_n = jnp.float32(1.0 / hidden)
                mean = jnp.sum(s) * inv_n
                var = jnp.sum(q) * inv_n - mean * mean
                rstd = _scalar_rsqrt(var + jnp.float32(1e-5))
                for h in range(n_vregs):
                    sl = pl.ds(h * _LANES, _LANES)
                    rows_v[t, sl] = (x[h] - mean) * rstd * g_v[sl] + b_v[sl]
                return c2

            lax.fori_loop(0, _CHUNK, tok_body, 0)
            pltpu.sync_copy(rows_v, out_hbm.at[pl.ds(base, _CHUNK)])
            return carry

        lax.fori_loop(0, n_chunks, chunk_body, 0)

    return body


def kernel(input_ids, token_type_ids, word_emb, pos_emb, type_emb,
           ln_gamma, ln_beta):
    batch, seq = input_ids.shape
    vocab, hidden = word_emb.shape
    tok = batch * seq

    ptab = (pos_emb[:seq, None, :] + type_emb[None, :, :]).reshape(
        2 * seq, hidden)
    pos_ids = jnp.arange(seq, dtype=jnp.int32)
    cidx = (pos_ids[None, :] * 2
            + token_type_ids.astype(jnp.int32)).reshape(tok)
    ids = input_ids.astype(jnp.int32).reshape(tok)

    info = plsc.get_sparse_core_info()
    num_workers = info.num_cores * info.num_subcores

    sc = _make_sc_kernel(tok, hidden, num_workers)
    out = sc(ids, cidx, word_emb, ptab,
             ln_gamma.astype(jnp.float32), ln_beta.astype(jnp.float32))
    return out.reshape(batch, seq, hidden)


# speedup vs baseline: 2.1448x; 2.1448x over previous
import functools

import jax
import jax.numpy as jnp
from jax import lax
from jax.experimental import pallas as pl
from jax.experimental.pallas import tpu as pltpu
from jax.experimental.pallas import tpu_sc as plsc

_HIDDEN = 128
_LANES = 16
_CHUNK = 128


def _scalar_rsqrt(v):
    i = lax.bitcast_convert_type(v, jnp.int32)
    i = jnp.int32(0x5F3759DF) - (i >> 1)
    y = lax.bitcast_convert_type(i, jnp.float32)
    for _ in range(3):
        y = y * (1.5 - 0.5 * v * y * y)
    return y


def _make_sc_kernel(tok, hidden, num_workers):
    per_w = tok // num_workers
    n_chunks = per_w // _CHUNK
    n_vregs = hidden // _LANES
    mesh = plsc.VectorSubcoreMesh(core_axis_name="c", subcore_axis_name="s")

    @functools.partial(
        pl.kernel,
        mesh=mesh,
        compiler_params=pltpu.CompilerParams(needs_layout_passes=False),
        out_type=jax.ShapeDtypeStruct((tok, hidden), jnp.float32),
        scratch_types=[
            pltpu.VMEM((_CHUNK,), jnp.int32),
            pltpu.VMEM((_CHUNK,), jnp.int32),
            pltpu.VMEM((_CHUNK, hidden), jnp.float32),
            pltpu.VMEM((_CHUNK, hidden), jnp.float32),
            pltpu.VMEM((hidden,), jnp.float32),
            pltpu.VMEM((hidden,), jnp.float32),
            pltpu.SemaphoreType.DMA,
            pltpu.SemaphoreType.DMA,
        ],
    )
    def body(ids_hbm, cidx_hbm, wemb_hbm, ptab_hbm, g_hbm, b_hbm, out_hbm,
             idx_v, cidx_v, rows_v, pt_v, g_v, b_v, sem_w, sem_p):
        wid = lax.axis_index("s") * 2 + lax.axis_index("c")
        base0 = wid * per_w
        pltpu.sync_copy(g_hbm, g_v)
        pltpu.sync_copy(b_hbm, b_v)

        def chunk_body(ci, carry):
            base = base0 + ci * _CHUNK
            pltpu.sync_copy(ids_hbm.at[pl.ds(base, _CHUNK)], idx_v)
            pltpu.sync_copy(cidx_hbm.at[pl.ds(base, _CHUNK)], cidx_v)
            cp_w = pltpu.async_copy(wemb_hbm.at[idx_v], rows_v, sem_w)
            cp_p = pltpu.async_copy(ptab_hbm.at[cidx_v], pt_v, sem_p)
            cp_w.wait()
            cp_p.wait()

            def tok_body(t, c2):
                x = [rows_v[t, pl.ds(h * _LANES, _LANES)]
                     + pt_v[t, pl.ds(h * _LANES, _LANES)]
                     for h in range(n_vregs)]
                s = x[0]
                q = x[0] * x[0]
                for h in range(1, n_vregs):
                    s = s + x[h]
                    q = q + x[h] * x[h]
                inv_n = jnp.float32(1.0 / hidden)
                mean = jnp.sum(s) * inv_n
                var = jnp.sum(q) * inv_n - mean * mean
                rstd = _scalar_rsqrt(var + jnp.float32(1e-5))
                for h in range(n_vregs):
                    sl = pl.ds(h * _LANES, _LANES)
                    rows_v[t, sl] = (x[h] - mean) * rstd * g_v[sl] + b_v[sl]
                return c2

            lax.fori_loop(0, _CHUNK, tok_body, 0)
            pltpu.sync_copy(rows_v, out_hbm.at[pl.ds(base, _CHUNK)])
            return carry

        lax.fori_loop(0, n_chunks, chunk_body, 0)

    return body


def kernel(input_ids, token_type_ids, word_emb, pos_emb, type_emb,
           ln_gamma, ln_beta):
    batch, seq = input_ids.shape
    vocab, hidden = word_emb.shape
    tok = batch * seq

    ptab = (pos_emb[:seq, None, :] + type_emb[None, :, :]).reshape(
        2 * seq, hidden)
    pos_ids = jnp.arange(seq, dtype=jnp.int32)
    cidx = (pos_ids[None, :] * 2
            + token_type_ids.astype(jnp.int32)).reshape(tok)
    ids = input_ids.astype(jnp.int32).reshape(tok)

    info = plsc.get_sparse_core_info()
    num_workers = info.num_cores * info.num_subcores

    sc = _make_sc_kernel(tok, hidden, num_workers)
    out = sc(ids, cidx, word_emb, ptab,
             ln_gamma.astype(jnp.float32), ln_beta.astype(jnp.float32))
    return out.reshape(batch, seq, hidden)


# speedup vs baseline: 2.1989x; 1.0252x over previous
import functools

import jax
import jax.numpy as jnp
from jax import lax
from jax.experimental import pallas as pl
from jax.experimental.pallas import tpu as pltpu
from jax.experimental.pallas import tpu_sc as plsc

_LANES = 16
_CHUNK = 128


def _scalar_rsqrt(v):
    i = lax.bitcast_convert_type(v, jnp.int32)
    i = jnp.int32(0x5F3759DF) - (i >> 1)
    y = lax.bitcast_convert_type(i, jnp.float32)
    for _ in range(3):
        y = y * (1.5 - 0.5 * v * y * y)
    return y


def _make_sc_kernel(tok, hidden, seq, num_workers):
    per_w = tok // num_workers
    n_chunks = per_w // _CHUNK
    n_vregs = hidden // _LANES
    assert per_w % seq == 0
    assert n_chunks % 2 == 0 and n_chunks >= 4
    mesh = plsc.VectorSubcoreMesh(core_axis_name="c", subcore_axis_name="s")

    @functools.partial(
        pl.kernel,
        mesh=mesh,
        compiler_params=pltpu.CompilerParams(needs_layout_passes=False),
        out_type=jax.ShapeDtypeStruct((tok, hidden), jnp.float32),
        scratch_types=[
            pltpu.VMEM((2, _CHUNK), jnp.int32),
            pltpu.VMEM((2, _CHUNK), jnp.float32),
            pltpu.VMEM((2, _CHUNK, hidden), jnp.float32),
            pltpu.VMEM((2, _CHUNK, hidden), jnp.float32),
            pltpu.VMEM((seq, hidden), jnp.float32),
            pltpu.VMEM((hidden,), jnp.float32),
            pltpu.VMEM((hidden,), jnp.float32),
            pltpu.VMEM((hidden,), jnp.float32),
            pltpu.SemaphoreType.DMA,
            pltpu.SemaphoreType.DMA,
            pltpu.SemaphoreType.DMA,
            pltpu.SemaphoreType.DMA,
        ],
    )
    def body(ids_hbm, ttf_hbm, wemb_hbm, ptab_hbm, td_hbm, g_hbm, b_hbm,
             out_hbm, idx_v, ttf_v, rows_v, outb_v, ptab_v, td_v, g_v, b_v,
             sem_g0, sem_g1, sem_o0, sem_o1):
        wid = lax.axis_index("s") * 2 + lax.axis_index("c")
        base0 = wid * per_w
        pltpu.sync_copy(ptab_hbm, ptab_v)
        pltpu.sync_copy(td_hbm, td_v)
        pltpu.sync_copy(g_hbm, g_v)
        pltpu.sync_copy(b_hbm, b_v)

        sem_g = (sem_g0, sem_g1)
        sem_o = (sem_o0, sem_o1)
        inv_n = jnp.float32(1.0 / hidden)

        def start_gather(c, slot):
            base = base0 + c * _CHUNK
            pltpu.sync_copy(ids_hbm.at[pl.ds(base, _CHUNK)], idx_v.at[slot])
            pltpu.sync_copy(ttf_hbm.at[pl.ds(base, _CHUNK)], ttf_v.at[slot])
            pltpu.async_copy(wemb_hbm.at[idx_v.at[slot]], rows_v.at[slot],
                             sem_g[slot])

        def wait_gather(slot):
            pltpu.make_async_copy(wemb_hbm.at[idx_v.at[slot]],
                                  rows_v.at[slot], sem_g[slot]).wait()

        def start_out(c, slot):
            base = base0 + c * _CHUNK
            pltpu.async_copy(outb_v.at[slot],
                             out_hbm.at[pl.ds(base, _CHUNK)], sem_o[slot])

        def wait_out(slot):
            pltpu.make_async_copy(outb_v.at[slot],
                                  out_hbm.at[pl.ds(base0, _CHUNK)],
                                  sem_o[slot]).wait()

        def compute_chunk(c, slot):
            rows = rows_v.at[slot]
            outb = outb_v.at[slot]
            ttf = ttf_v.at[slot]
            s0 = lax.rem(c * _CHUNK, seq)

            def tok_body(t, sp):
                tfac = plsc.load_gather(
                    ttf, [jnp.full((_LANES,), 0, jnp.int32) + t])
                x = []
                for h in range(n_vregs):
                    sl = pl.ds(h * _LANES, _LANES)
                    x.append(rows[t, sl] + ptab_v[sp, sl] + tfac * td_v[sl])
                s = x[0]
                q = x[0] * x[0]
                for h in range(1, n_vregs):
                    s = s + x[h]
                    q = q + x[h] * x[h]
                mean = jnp.sum(s) * inv_n
                var = jnp.sum(q) * inv_n - mean * mean
                rstd = _scalar_rsqrt(var + jnp.float32(1e-5))
                shift = mean * rstd
                for h in range(n_vregs):
                    sl = pl.ds(h * _LANES, _LANES)
                    outb[t, sl] = (x[h] * rstd - shift) * g_v[sl] + b_v[sl]
                sp = sp + 1
                return jnp.where(sp >= seq, sp - seq, sp)

            lax.fori_loop(0, _CHUNK, tok_body, s0, unroll=2)

        start_gather(0, 0)

        start_gather(1, 1)
        wait_gather(0)
        compute_chunk(0, 0)
        start_out(0, 0)
        start_gather(2, 0)
        wait_gather(1)
        compute_chunk(1, 1)
        start_out(1, 1)

        def group_body(g, carry):
            c0 = 2 * g
            start_gather(c0 + 1, 1)
            wait_gather(0)
            wait_out(0)
            compute_chunk(c0, 0)
            start_out(c0, 0)

            start_gather(c0 + 2, 0)
            wait_gather(1)
            wait_out(1)
            compute_chunk(c0 + 1, 1)
            start_out(c0 + 1, 1)
            return carry

        lax.fori_loop(1, n_chunks // 2 - 1, group_body, 0)

        c0 = n_chunks - 2
        start_gather(c0 + 1, 1)
        wait_gather(0)
        wait_out(0)
        compute_chunk(c0, 0)
        start_out(c0, 0)

        wait_gather(1)
        wait_out(1)
        compute_chunk(c0 + 1, 1)
        start_out(c0 + 1, 1)

        wait_out(0)
        wait_out(1)

    return body


def kernel(input_ids, token_type_ids, word_emb, pos_emb, type_emb,
           ln_gamma, ln_beta):
    batch, seq = input_ids.shape
    vocab, hidden = word_emb.shape
    tok = batch * seq

    ptab = pos_emb[:seq] + type_emb[0][None, :]
    tdelta = type_emb[1] - type_emb[0]
    ttf = token_type_ids.astype(jnp.float32).reshape(tok)
    ids = input_ids.astype(jnp.int32).reshape(tok)

    info = plsc.get_sparse_core_info()
    num_workers = info.num_cores * info.num_subcores

    sc = _make_sc_kernel(tok, hidden, seq, num_workers)
    out = sc(ids, ttf, word_emb, ptab, tdelta,
             ln_gamma.astype(jnp.float32), ln_beta.astype(jnp.float32))
    return out.reshape(batch, seq, hidden)
